# baseline (speedup 1.0000x reference)
import functools

import jax
import jax.numpy as jnp
from jax import lax
from jax.experimental import pallas as pl
from jax.experimental.pallas import tpu as pltpu
from jax.experimental.pallas import tpu_sc as plsc

BATCH = 4096
SEQ = 200
DIM = 64
NC, NS = 2, 16
NW = NC * NS
CHUNK = 128
LANES = 16
NBUF = 4
DBLK = DIM // 8
BBLK = BATCH // CHUNK
QUARTERS = DIM // LANES


def _sc_body(ids_hbm, tok_hbm, pos_hbm, out_hbm, idx_v, g_v, t_v, pos_v,
             gsems, ssems):
    wid = lax.axis_index("s") * NC + lax.axis_index("c")

    pltpu.sync_copy(pos_hbm, pos_v)
    pltpu.sync_copy(ids_hbm.at[:, pl.ds(wid * CHUNK, CHUNK)], idx_v)

    def fire_gather(s, b):
        pltpu.async_copy(tok_hbm.at[idx_v.at[s]], g_v.at[b], gsems.at[b])

    def wait_gather(b):
        pltpu.make_async_copy(tok_hbm.at[idx_v.at[0]], g_v.at[b],
                              gsems.at[b]).wait()

    def fire_store(s, b):
        pltpu.async_copy(t_v.at[b], out_hbm.at[s, :, wid], ssems.at[b])

    def wait_store(b):
        pltpu.make_async_copy(t_v.at[b], out_hbm.at[0, :, 0],
                              ssems.at[b]).wait()

    iota = lax.iota(jnp.int32, LANES)
    rows_c = [(iota + c * LANES) // 8 for c in range(QUARTERS)]
    cols_c = [((iota + c * LANES) % 8) * CHUNK for c in range(QUARTERS)]

    fire_gather(0, 0)
    fire_gather(1, 1)

    @pl.loop(0, SEQ, step=NBUF)
    def _grp(s0):
        for u in range(NBUF):
            b = u
            s = s0 + u
            bn = (u + 2) % NBUF

            @pl.when(s + 2 < SEQ)
            def _():
                @pl.when(s >= 2)
                def _():
                    wait_store(bn)
                fire_gather(s + 2, bn)

            wait_gather(b)
            pq = [pos_v[s, pl.ds(c * LANES, LANES)] for c in range(QUARTERS)]

            @pl.loop(0, CHUNK, unroll=8)
            def _kloop(k):
                for c in range(QUARTERS):
                    x = g_v[b, k, pl.ds(c * LANES, LANES)] + pq[c]
                    plsc.store_scatter(t_v.at[b], [rows_c[c], cols_c[c] + k],
                                       x)

            fire_store(s, b)

    for b in range(NBUF):
        wait_store(b)


@functools.partial(jax.jit, static_argnames=())
def kernel(input_ids, token_table, pos_table):
    ids_t = input_ids.T.astype(jnp.int32)
    mesh = plsc.VectorSubcoreMesh(core_axis_name="c", subcore_axis_name="s")
    out5 = pl.kernel(
        _sc_body,
        out_type=jax.ShapeDtypeStruct((SEQ, DBLK, BBLK, 8 * CHUNK),
                                      jnp.float32),
        mesh=mesh,
        compiler_params=pltpu.CompilerParams(use_tc_tiling_on_sc=False,
                                             needs_layout_passes=False),
        scratch_types=[
            pltpu.VMEM((SEQ, CHUNK), jnp.int32),
            pltpu.VMEM((NBUF, CHUNK, DIM), jnp.float32),
            pltpu.VMEM((NBUF, DBLK, 8 * CHUNK), jnp.float32),
            pltpu.VMEM((SEQ, DIM), jnp.float32),
            pltpu.SemaphoreType.DMA((NBUF,)),
            pltpu.SemaphoreType.DMA((NBUF,)),
        ],
    )(ids_t, token_table, pos_table)
    out5 = out5.reshape(SEQ, DBLK, BBLK, 8, CHUNK)
    return out5.transpose(2, 4, 0, 1, 3).reshape(BATCH, SEQ, DIM)


# speedup vs baseline: 1.3865x; 1.3865x over previous
import functools

import jax
import jax.numpy as jnp
from jax import lax
from jax.experimental import pallas as pl
from jax.experimental.pallas import tpu as pltpu
from jax.experimental.pallas import tpu_sc as plsc

BATCH = 4096
SEQ = 200
DIM = 64
N = BATCH * SEQ
NC, NS = 2, 16
NW = NC * NS
SPW = BATCH // NW
LANES = 16
NBUF = 4


def _sc_body(ids_hbm, tok_hbm, pos_hbm, out_hbm, idx_v, g_v, pos_v,
             gsems, ssems):
    wid = lax.axis_index("s") * NC + lax.axis_index("c")

    pltpu.sync_copy(pos_hbm, pos_v)
    pltpu.sync_copy(ids_hbm.at[pl.ds(wid * SPW, SPW)], idx_v)

    def fire_gather(j, b):
        pltpu.async_copy(tok_hbm.at[idx_v.at[j]], g_v.at[b], gsems.at[b])

    def wait_gather(b):
        pltpu.make_async_copy(tok_hbm.at[idx_v.at[0]], g_v.at[b],
                              gsems.at[b]).wait()

    def fire_store(j, b):
        base = (wid * SPW + j) * SEQ
        pltpu.async_copy(g_v.at[b], out_hbm.at[pl.ds(base, SEQ)], ssems.at[b])

    def wait_store(b):
        pltpu.make_async_copy(g_v.at[b], out_hbm.at[pl.ds(0, SEQ)],
                              ssems.at[b]).wait()

    fire_gather(0, 0)
    fire_gather(1, 1)

    @pl.loop(0, SPW, step=NBUF)
    def _grp(j0):
        for u in range(NBUF):
            b = u
            j = j0 + u
            bn = (u + 2) % NBUF

            @pl.when(j + 2 < SPW)
            def _():
                @pl.when(j >= 2)
                def _():
                    wait_store(bn)
                fire_gather(j + 2, bn)

            wait_gather(b)

            @pl.loop(0, SEQ, unroll=8)
            def _row(r):
                for c in range(DIM // LANES):
                    sl = pl.ds(c * LANES, LANES)
                    plsc.addupdate(g_v.at[b, r, sl], pos_v[r, sl])

            fire_store(j, b)

    for b in range(NBUF):
        wait_store(b)


@functools.partial(jax.jit, static_argnames=())
def kernel(input_ids, token_table, pos_table):
    ids = input_ids.astype(jnp.int32)
    mesh = plsc.VectorSubcoreMesh(core_axis_name="c", subcore_axis_name="s")
    out = pl.kernel(
        _sc_body,
        out_type=jax.ShapeDtypeStruct((N, DIM), jnp.float32),
        mesh=mesh,
        compiler_params=pltpu.CompilerParams(use_tc_tiling_on_sc=False,
                                             needs_layout_passes=False),
        scratch_types=[
            pltpu.VMEM((SPW, SEQ), jnp.int32),
            pltpu.VMEM((NBUF, SEQ, DIM), jnp.float32),
            pltpu.VMEM((SEQ, DIM), jnp.float32),
            pltpu.SemaphoreType.DMA((NBUF,)),
            pltpu.SemaphoreType.DMA((NBUF,)),
        ],
    )(ids, token_table, pos_table)
    return out.reshape(BATCH, SEQ, DIM)


# speedup vs baseline: 1.4306x; 1.0319x over previous
import functools

import jax
import jax.numpy as jnp
from jax import lax
from jax.experimental import pallas as pl
from jax.experimental.pallas import tpu as pltpu
from jax.experimental.pallas import tpu_sc as plsc

BATCH = 4096
SEQ = 200
DIM = 64
PADW = 128
N = BATCH * SEQ
NC, NS = 2, 16
NW = NC * NS
SPW = BATCH // NW
LANES = 16
NBUF = 2


def _sc_body(ids_hbm, tok_hbm, pos_hbm, out_hbm, idx_v, g_v, pos_v,
             gsems, ssems):
    wid = lax.axis_index("s") * NC + lax.axis_index("c")

    pltpu.sync_copy(pos_hbm, pos_v)
    pltpu.sync_copy(ids_hbm.at[pl.ds(wid * SPW * SEQ, SPW * SEQ)], idx_v)

    def fire_gather(j, b):
        pltpu.async_copy(tok_hbm.at[idx_v.at[pl.ds(j * SEQ, SEQ)]],
                         g_v.at[b], gsems.at[b])

    def wait_gather(b):
        pltpu.make_async_copy(tok_hbm.at[idx_v.at[pl.ds(0, SEQ)]],
                              g_v.at[b], gsems.at[b]).wait()

    def fire_store(j, b):
        base = (wid * SPW + j) * SEQ
        pltpu.async_copy(g_v.at[b], out_hbm.at[pl.ds(base, SEQ)], ssems.at[b])

    def wait_store(b):
        pltpu.make_async_copy(g_v.at[b], out_hbm.at[pl.ds(0, SEQ)],
                              ssems.at[b]).wait()

    fire_gather(0, 0)

    @pl.loop(0, SPW, step=NBUF)
    def _grp(j0):
        for u in range(NBUF):
            b = u
            j = j0 + u
            bn = (u + 1) % NBUF

            @pl.when(j + 1 < SPW)
            def _():
                @pl.when(j >= 1)
                def _():
                    wait_store(bn)
                fire_gather(j + 1, bn)

            wait_gather(b)

            @pl.loop(0, SEQ, unroll=8)
            def _row(r):
                pbase = (r % 2) * DIM
                for c in range(DIM // LANES):
                    plsc.addupdate(g_v.at[b, r, pl.ds(c * LANES, LANES)],
                                   pos_v[r // 2, pl.ds(pbase + c * LANES,
                                                       LANES)])

            fire_store(j, b)

    for b in range(NBUF):
        wait_store(b)


@functools.partial(jax.jit, static_argnames=())
def kernel(input_ids, token_table, pos_table):
    ids = input_ids.reshape(N).astype(jnp.int32)
    tok_p = jnp.pad(token_table, ((0, 0), (0, PADW - DIM)))
    pos2 = pos_table.reshape(SEQ // 2, 2 * DIM)
    mesh = plsc.VectorSubcoreMesh(core_axis_name="c", subcore_axis_name="s")
    out_p = pl.kernel(
        _sc_body,
        out_type=jax.ShapeDtypeStruct((N, PADW), jnp.float32),
        mesh=mesh,
        scratch_types=[
            pltpu.VMEM((SPW * SEQ,), jnp.int32),
            pltpu.VMEM((NBUF, SEQ, PADW), jnp.float32),
            pltpu.VMEM((SEQ // 2, 2 * DIM), jnp.float32),
            pltpu.SemaphoreType.DMA((NBUF,)),
            pltpu.SemaphoreType.DMA((NBUF,)),
        ],
    )(ids, tok_p, pos2)
    return out_p.reshape(BATCH, SEQ, PADW)[:, :, :DIM]


# speedup vs baseline: 1.5902x; 1.1116x over previous
import functools

import jax
import jax.numpy as jnp
from jax import lax
from jax.experimental import pallas as pl
from jax.experimental.pallas import tpu as pltpu
from jax.experimental.pallas import tpu_sc as plsc

BATCH = 4096
SEQ = 200
DIM = 64
PADW = 128
N = BATCH * SEQ
NC, NS = 2, 16
NW = NC * NS
SPW = BATCH // NW
LANES = 16
NBUF = 3


def _sc_body(ids_hbm, tok_hbm, pos_hbm, out_hbm, idx_v, g_v, pos_v,
             gsems, ssems):
    wid = lax.axis_index("s") * NC + lax.axis_index("c")

    pltpu.sync_copy(pos_hbm, pos_v)
    pltpu.sync_copy(ids_hbm.at[pl.ds(wid * SPW * SEQ, SPW * SEQ)], idx_v)

    def fire_gather(j, b):
        pltpu.async_copy(tok_hbm.at[idx_v.at[pl.ds(j * SEQ, SEQ)]],
                         g_v.at[b], gsems.at[b])

    def wait_gather(b):
        pltpu.make_async_copy(tok_hbm.at[idx_v.at[pl.ds(0, SEQ)]],
                              g_v.at[b], gsems.at[b]).wait()

    def fire_store(j, b):
        base = (wid * SPW + j) * SEQ
        pltpu.async_copy(g_v.at[b], out_hbm.at[pl.ds(base, SEQ)], ssems.at[b])

    def wait_store(b):
        pltpu.make_async_copy(g_v.at[b], out_hbm.at[pl.ds(0, SEQ)],
                              ssems.at[b]).wait()

    fire_gather(0, 0)

    @pl.loop(0, SPW + (-SPW) % NBUF, step=NBUF)
    def _grp(j0):
        for u in range(NBUF):
            b = u
            j = j0 + u

            @pl.when(j < SPW)
            def _():
                bn = (u + 1) % NBUF

                @pl.when(j + 1 < SPW)
                def _():
                    @pl.when(j >= 2)
                    def _():
                        wait_store(bn)
                    fire_gather(j + 1, bn)

                wait_gather(b)

                @pl.loop(0, SEQ, unroll=8)
                def _row(r):
                    pbase = (r % 2) * DIM
                    for c in range(DIM // LANES):
                        plsc.addupdate(
                            g_v.at[b, r, pl.ds(c * LANES, LANES)],
                            pos_v[r // 2, pl.ds(pbase + c * LANES, LANES)])

                fire_store(j, b)

    for b in range(NBUF):
        wait_store(b)


@functools.partial(jax.jit, static_argnames=())
def kernel(input_ids, token_table, pos_table):
    ids = input_ids.reshape(N).astype(jnp.int32)
    tok_p = jnp.pad(token_table, ((0, 0), (0, PADW - DIM)))
    pos2 = pos_table.reshape(SEQ // 2, 2 * DIM)
    mesh = plsc.VectorSubcoreMesh(core_axis_name="c", subcore_axis_name="s")
    out_p = pl.kernel(
        _sc_body,
        out_type=jax.ShapeDtypeStruct((N, PADW), jnp.float32),
        mesh=mesh,
        scratch_types=[
            pltpu.VMEM((SPW * SEQ,), jnp.int32),
            pltpu.VMEM((NBUF, SEQ, PADW), jnp.float32),
            pltpu.VMEM((SEQ // 2, 2 * DIM), jnp.float32),
            pltpu.SemaphoreType.DMA((NBUF,)),
            pltpu.SemaphoreType.DMA((NBUF,)),
        ],
    )(ids, tok_p, pos2)
    return out_p.reshape(BATCH, SEQ, PADW)[:, :, :DIM]
